# Initial kernel scaffold; baseline (speedup 1.0000x reference)
import jax
import jax.numpy as jnp
from jax.experimental import pallas as pl


def kernel(h, edge_index, W1, b1, W2, b2):
    raise NotImplementedError("write your pallas kernel here")


# speedup vs baseline: 3.8057x; 3.8057x over previous
import functools

import jax
import jax.numpy as jnp
from jax import lax
from jax.experimental import pallas as pl
from jax.experimental.pallas import tpu as pltpu
from jax.experimental.pallas import tpu_sc as plsc

N = 10000
E = 320000
D = 128
NC = 2
NS = 16
NW = NC * NS
EPW = E // NW
B = 80
NCHUNK = EPW // B
NPAD = 10240
RPS = NPAD // NS
ZR = 128


def _deg_body(src_hbm, dst_hbm, degs_hbm, degd_hbm,
              ones_v, sidx_v, didx_v, zrow_v, accs, accd):
    c = lax.axis_index("c")
    s = lax.axis_index("s")
    wid = c * NS + s
    base = wid * EPW
    o16 = jnp.full((16,), 1.0, jnp.float32)
    z16 = jnp.zeros((16,), jnp.float32)

    def init_ones(i, _):
        ones_v[i, :] = o16
        return 0
    lax.fori_loop(0, B, init_ones, 0)

    def init_z(i, _):
        zrow_v[i, :] = z16
        return 0
    lax.fori_loop(0, ZR, init_z, 0)

    def zloop(k, _):
        start = s * RPS + k * ZR
        pltpu.sync_copy(zrow_v, accs.at[pl.ds(start, ZR)])
        pltpu.sync_copy(zrow_v, accd.at[pl.ds(start, ZR)])
        return 0
    lax.fori_loop(0, RPS // ZR, zloop, 0)
    plsc.subcore_barrier()

    def eloop(j, _):
        off = base + j * B
        pltpu.sync_copy(src_hbm.at[pl.ds(off, B)], sidx_v)
        pltpu.sync_copy(dst_hbm.at[pl.ds(off, B)], didx_v)
        pltpu.sync_copy(ones_v, accs.at[sidx_v], add=True)
        pltpu.sync_copy(ones_v, accd.at[didx_v], add=True)
        return 0
    lax.fori_loop(0, NCHUNK, eloop, 0)
    plsc.subcore_barrier()

    r0 = s * RPS
    pltpu.sync_copy(accs.at[pl.ds(r0, RPS)], degs_hbm.at[c, pl.ds(r0, RPS), :])
    pltpu.sync_copy(accd.at[pl.ds(r0, RPS)], degd_hbm.at[c, pl.ds(r0, RPS), :])


@functools.cache
def _deg_call():
    mesh = plsc.VectorSubcoreMesh(core_axis_name="c", subcore_axis_name="s",
                                  num_cores=NC, num_subcores=NS)
    return pl.kernel(
        _deg_body,
        out_type=(jax.ShapeDtypeStruct((NC, NPAD, 16), jnp.float32),
                  jax.ShapeDtypeStruct((NC, NPAD, 16), jnp.float32)),
        mesh=mesh,
        scratch_types=[
            pltpu.VMEM((B, 16), jnp.float32),
            pltpu.VMEM((B,), jnp.int32),
            pltpu.VMEM((B,), jnp.int32),
            pltpu.VMEM((ZR, 16), jnp.float32),
            pltpu.VMEM_SHARED((NPAD, 16), jnp.float32),
            pltpu.VMEM_SHARED((NPAD, 16), jnp.float32),
        ],
    )


def _agg_body(x_hbm, src_hbm, dst_hbm, aggp_hbm,
              rows_v, sidx_v, didx_v, zrow_v, acc, sem):
    c = lax.axis_index("c")
    s = lax.axis_index("s")
    wid = c * NS + s
    base = wid * EPW
    z16 = jnp.zeros((16,), jnp.float32)

    def init_z(i, _):
        for jj in range(D // 16):
            zrow_v[i, pl.ds(jj * 16, 16)] = z16
        return 0
    lax.fori_loop(0, ZR, init_z, 0)

    def zloop(k, _):
        start = s * RPS + k * ZR
        pltpu.sync_copy(zrow_v, acc.at[pl.ds(start, ZR)])
        return 0
    lax.fori_loop(0, RPS // ZR, zloop, 0)
    plsc.subcore_barrier()

    def eloop(j, _):
        off = base + j * B
        pltpu.sync_copy(src_hbm.at[pl.ds(off, B)], sidx_v)
        pltpu.sync_copy(dst_hbm.at[pl.ds(off, B)], didx_v)
        pltpu.async_copy(x_hbm.at[sidx_v], rows_v, sem).wait()
        pltpu.sync_copy(rows_v, acc.at[didx_v], add=True)
        return 0
    lax.fori_loop(0, NCHUNK, eloop, 0)
    plsc.subcore_barrier()

    r0 = s * RPS
    pltpu.sync_copy(acc.at[pl.ds(r0, RPS)], aggp_hbm.at[c, pl.ds(r0, RPS), :])


@functools.cache
def _agg_call():
    mesh = plsc.VectorSubcoreMesh(core_axis_name="c", subcore_axis_name="s",
                                  num_cores=NC, num_subcores=NS)
    return pl.kernel(
        _agg_body,
        out_type=jax.ShapeDtypeStruct((NC, NPAD, D), jnp.float32),
        mesh=mesh,
        scratch_types=[
            pltpu.VMEM((B, D), jnp.float32),
            pltpu.VMEM((B,), jnp.int32),
            pltpu.VMEM((B,), jnp.int32),
            pltpu.VMEM((ZR, D), jnp.float32),
            pltpu.VMEM_SHARED((NPAD, D), jnp.float32),
            pltpu.SemaphoreType.DMA,
        ],
    )


def _prep_body(h_ref, degs_ref, degd_ref, x0_ref, oisq_ref, iisq_ref):
    ds = (degs_ref[0] + degs_ref[1])[:N, 0:1]
    dd = (degd_ref[0] + degd_ref[1])[:N, 0:1]
    oisq = lax.rsqrt(jnp.maximum(ds, 1.0))
    iisq = lax.rsqrt(jnp.maximum(dd, 1.0))
    oisq_ref[...] = oisq
    iisq_ref[...] = iisq
    x0_ref[...] = h_ref[...] * oisq


_prep_call = pl.pallas_call(
    _prep_body,
    out_shape=(jax.ShapeDtypeStruct((N, D), jnp.float32),
               jax.ShapeDtypeStruct((N, 1), jnp.float32),
               jax.ShapeDtypeStruct((N, 1), jnp.float32)),
)


def _l1_body(aggp_ref, oisq_ref, iisq_ref, w_ref, b_ref, x1_ref):
    agg = (aggp_ref[0] + aggp_ref[1])[:N] * iisq_ref[...]
    h1 = jnp.dot(agg, w_ref[...], preferred_element_type=jnp.float32) + b_ref[...]
    x1_ref[...] = jnp.maximum(h1, 0.0) * oisq_ref[...]


_l1_call = pl.pallas_call(
    _l1_body,
    out_shape=jax.ShapeDtypeStruct((N, D), jnp.float32),
)


def _l2_body(aggp_ref, iisq_ref, w_ref, b_ref, p_ref, h2_ref):
    agg = (aggp_ref[0] + aggp_ref[1])[:N] * iisq_ref[...]
    h2 = jnp.dot(agg, w_ref[...], preferred_element_type=jnp.float32) + b_ref[...]
    m = jnp.max(h2, axis=1, keepdims=True)
    e = jnp.exp(h2 - m)
    p_ref[...] = e / jnp.sum(e, axis=1, keepdims=True)
    h2_ref[...] = h2


_l2_call = pl.pallas_call(
    _l2_body,
    out_shape=(jax.ShapeDtypeStruct((N, D), jnp.float32),
               jax.ShapeDtypeStruct((N, D), jnp.float32)),
)


def kernel(h, edge_index, W1, b1, W2, b2):
    src = edge_index[0]
    dst = edge_index[1]
    degs, degd = _deg_call()(src, dst)
    x0, oisq, iisq = _prep_call(h, degs, degd)
    aggp1 = _agg_call()(x0, src, dst)
    x1 = _l1_call(aggp1, oisq, iisq, W1, b1.reshape(1, D))
    aggp2 = _agg_call()(x1, src, dst)
    p, h2 = _l2_call(aggp2, iisq, W2, b2.reshape(1, D))
    return (p, h2)


# speedup vs baseline: 5.8722x; 1.5430x over previous
import functools

import jax
import jax.numpy as jnp
from jax import lax
from jax.experimental import pallas as pl
from jax.experimental.pallas import tpu as pltpu
from jax.experimental.pallas import tpu_sc as plsc

N = 10000
E = 320000
D = 128
NC = 2
NS = 16
NW = NC * NS
EPW = E // NW
B = 80
NCHUNK = EPW // B
NPAD = 10240
RPS = NPAD // NS
ZR = 128


def _deg_body(src_hbm, dst_hbm, degs_hbm, degd_hbm,
              ones_v, sidx_v, didx_v, zrow_v, accs, accd):
    c = lax.axis_index("c")
    s = lax.axis_index("s")
    wid = c * NS + s
    base = wid * EPW
    o16 = jnp.full((16,), 1.0, jnp.float32)
    z16 = jnp.zeros((16,), jnp.float32)

    def init_ones(i, _):
        ones_v[i, :] = o16
        return 0
    lax.fori_loop(0, B, init_ones, 0)

    def init_z(i, _):
        zrow_v[i, :] = z16
        return 0
    lax.fori_loop(0, ZR, init_z, 0)

    def zloop(k, _):
        start = s * RPS + k * ZR
        pltpu.sync_copy(zrow_v, accs.at[pl.ds(start, ZR)])
        pltpu.sync_copy(zrow_v, accd.at[pl.ds(start, ZR)])
        return 0
    lax.fori_loop(0, RPS // ZR, zloop, 0)
    plsc.subcore_barrier()

    def eloop(j, _):
        off = base + j * B
        pltpu.sync_copy(src_hbm.at[pl.ds(off, B)], sidx_v)
        pltpu.sync_copy(dst_hbm.at[pl.ds(off, B)], didx_v)
        pltpu.sync_copy(ones_v, accs.at[sidx_v], add=True)
        pltpu.sync_copy(ones_v, accd.at[didx_v], add=True)
        return 0
    lax.fori_loop(0, NCHUNK, eloop, 0)
    plsc.subcore_barrier()

    r0 = s * RPS
    pltpu.sync_copy(accs.at[pl.ds(r0, RPS)], degs_hbm.at[c, pl.ds(r0, RPS), :])
    pltpu.sync_copy(accd.at[pl.ds(r0, RPS)], degd_hbm.at[c, pl.ds(r0, RPS), :])


@functools.cache
def _deg_call():
    mesh = plsc.VectorSubcoreMesh(core_axis_name="c", subcore_axis_name="s",
                                  num_cores=NC, num_subcores=NS)
    return pl.kernel(
        _deg_body,
        out_type=(jax.ShapeDtypeStruct((NC, NPAD, 16), jnp.float32),
                  jax.ShapeDtypeStruct((NC, NPAD, 16), jnp.float32)),
        mesh=mesh,
        scratch_types=[
            pltpu.VMEM((B, 16), jnp.float32),
            pltpu.VMEM((B,), jnp.int32),
            pltpu.VMEM((B,), jnp.int32),
            pltpu.VMEM((ZR, 16), jnp.float32),
            pltpu.VMEM_SHARED((NPAD, 16), jnp.float32),
            pltpu.VMEM_SHARED((NPAD, 16), jnp.float32),
        ],
    )


def _agg_body(x_hbm, src_hbm, dst_hbm, aggp_hbm,
              rows_a, rows_b, sidx_v, didx_v, acc, sem_a, sem_b):
    c = lax.axis_index("c")
    s = lax.axis_index("s")
    wid = c * NS + s
    z16 = jnp.zeros((16,), jnp.float32)

    pltpu.sync_copy(src_hbm.at[pl.ds(wid * EPW, EPW)], sidx_v)
    pltpu.sync_copy(dst_hbm.at[wid], didx_v)

    def init_z(i, _):
        for jj in range(D // 16):
            rows_b[i, pl.ds(jj * 16, 16)] = z16
        return 0
    lax.fori_loop(0, B, init_z, 0)

    def zloop(k, _):
        start = s * RPS + k * B
        pltpu.sync_copy(rows_b, acc.at[pl.ds(start, B)])
        return 0
    lax.fori_loop(0, RPS // B, zloop, 0)
    plsc.subcore_barrier()

    def eloop(i, _):
        j = i * 2
        ha = pltpu.async_copy(x_hbm.at[sidx_v.at[pl.ds(j * B, B)]],
                              rows_a, sem_a)
        hb = pltpu.async_copy(x_hbm.at[sidx_v.at[pl.ds((j + 1) * B, B)]],
                              rows_b, sem_b)
        ha.wait()
        pltpu.sync_copy(rows_a, acc.at[didx_v.at[j]], add=True)
        hb.wait()
        pltpu.sync_copy(rows_b, acc.at[didx_v.at[j + 1]], add=True)
        return 0
    lax.fori_loop(0, NCHUNK // 2, eloop, 0)
    pltpu.async_copy(x_hbm.at[sidx_v.at[pl.ds((NCHUNK - 1) * B, B)]],
                     rows_a, sem_a).wait()
    pltpu.sync_copy(rows_a, acc.at[didx_v.at[NCHUNK - 1]], add=True)
    plsc.subcore_barrier()

    r0 = s * RPS
    pltpu.sync_copy(acc.at[pl.ds(r0, RPS)], aggp_hbm.at[c, pl.ds(r0, RPS), :])


@functools.cache
def _agg_call():
    mesh = plsc.VectorSubcoreMesh(core_axis_name="c", subcore_axis_name="s",
                                  num_cores=NC, num_subcores=NS)
    return pl.kernel(
        _agg_body,
        out_type=jax.ShapeDtypeStruct((NC, NPAD, D), jnp.float32),
        mesh=mesh,
        scratch_types=[
            pltpu.VMEM((B, D), jnp.float32),
            pltpu.VMEM((B, D), jnp.float32),
            pltpu.VMEM((EPW,), jnp.int32),
            pltpu.VMEM((NCHUNK, B), jnp.int32),
            pltpu.VMEM_SHARED((NPAD, D), jnp.float32),
            pltpu.SemaphoreType.DMA,
            pltpu.SemaphoreType.DMA,
        ],
    )


def _prep_body(h_ref, degs_ref, degd_ref, x0_ref, oisq_ref, iisq_ref):
    ds = (degs_ref[0] + degs_ref[1])[:N, 0:1]
    dd = (degd_ref[0] + degd_ref[1])[:N, 0:1]
    oisq = lax.rsqrt(jnp.maximum(ds, 1.0))
    iisq = lax.rsqrt(jnp.maximum(dd, 1.0))
    oisq_ref[...] = oisq
    iisq_ref[...] = iisq
    x0_ref[...] = h_ref[...] * oisq


_prep_call = pl.pallas_call(
    _prep_body,
    out_shape=(jax.ShapeDtypeStruct((N, D), jnp.float32),
               jax.ShapeDtypeStruct((N, 1), jnp.float32),
               jax.ShapeDtypeStruct((N, 1), jnp.float32)),
)


def _l1_body(aggp_ref, oisq_ref, iisq_ref, w_ref, b_ref, x1_ref):
    agg = (aggp_ref[0] + aggp_ref[1])[:N] * iisq_ref[...]
    h1 = jnp.dot(agg, w_ref[...], preferred_element_type=jnp.float32) + b_ref[...]
    x1_ref[...] = jnp.maximum(h1, 0.0) * oisq_ref[...]


_l1_call = pl.pallas_call(
    _l1_body,
    out_shape=jax.ShapeDtypeStruct((N, D), jnp.float32),
)


def _l2_body(aggp_ref, iisq_ref, w_ref, b_ref, p_ref, h2_ref):
    agg = (aggp_ref[0] + aggp_ref[1])[:N] * iisq_ref[...]
    h2 = jnp.dot(agg, w_ref[...], preferred_element_type=jnp.float32) + b_ref[...]
    m = jnp.max(h2, axis=1, keepdims=True)
    e = jnp.exp(h2 - m)
    p_ref[...] = e / jnp.sum(e, axis=1, keepdims=True)
    h2_ref[...] = h2


_l2_call = pl.pallas_call(
    _l2_body,
    out_shape=(jax.ShapeDtypeStruct((N, D), jnp.float32),
               jax.ShapeDtypeStruct((N, D), jnp.float32)),
)


def kernel(h, edge_index, W1, b1, W2, b2):
    src1d = edge_index[0]
    dst1d = edge_index[1]
    dst3d = dst1d.reshape(NW, NCHUNK, B)
    degs, degd = _deg_call()(src1d, dst1d)
    x0, oisq, iisq = _prep_call(h, degs, degd)
    aggp1 = _agg_call()(x0, src1d, dst3d)
    x1 = _l1_call(aggp1, oisq, iisq, W1, b1.reshape(1, D))
    aggp2 = _agg_call()(x1, src1d, dst3d)
    p, h2 = _l2_call(aggp2, iisq, W2, b2.reshape(1, D))
    return (p, h2)
